# baseline (speedup 1.0000x reference)
import functools

import jax
import jax.numpy as jnp
from jax import lax
from jax.experimental import pallas as pl
from jax.experimental.pallas import tpu as pltpu
from jax.experimental.pallas import tpu_sc as plsc

B = 16384
D = 32

_info = plsc.get_sparse_core_info()
_NC, _NS = _info.num_cores, _info.num_subcores
NW = _NC * _NS
BPW = B // NW
CH = 128
NCH = BPW // CH


def _sc_gather(users, items, mf_u, mf_i, mlp_u, mlp_i):
    mesh = plsc.VectorSubcoreMesh(core_axis_name="c", subcore_axis_name="s")

    @functools.partial(
        pl.kernel, mesh=mesh,
        out_type=[jax.ShapeDtypeStruct((B, D), jnp.float32)] * 4,
        scratch_types=[
            pltpu.VMEM((BPW,), jnp.int32),
            pltpu.VMEM((BPW,), jnp.int32),
            pltpu.VMEM((BPW, D), jnp.float32),
            pltpu.VMEM((BPW, D), jnp.float32),
            pltpu.VMEM((BPW, D), jnp.float32),
            pltpu.VMEM((BPW, D), jnp.float32),
            pltpu.SemaphoreType.DMA,
        ],
        compiler_params=pltpu.CompilerParams(use_tc_tiling_on_sc=False),
    )
    def k(users_h, items_h, mfu_h, mfi_h, mlpu_h, mlpi_h,
          o_mfu, o_mfi, o_mlpu, o_mlpi,
          idx_u, idx_i, v_mfu, v_mfi, v_mlpu, v_mlpi, sem):
        wid = lax.axis_index("s") * _NC + lax.axis_index("c")
        base = wid * BPW
        pltpu.sync_copy(users_h.at[pl.ds(base, BPW)], idx_u)
        pltpu.sync_copy(items_h.at[pl.ds(base, BPW)], idx_i)
        copies = []
        for c in range(NCH):
            s = pl.ds(c * CH, CH)
            copies.append(pltpu.async_copy(mfu_h.at[idx_u.at[s]], v_mfu.at[s], sem))
            copies.append(pltpu.async_copy(mfi_h.at[idx_i.at[s]], v_mfi.at[s], sem))
            copies.append(pltpu.async_copy(mlpu_h.at[idx_u.at[s]], v_mlpu.at[s], sem))
            copies.append(pltpu.async_copy(mlpi_h.at[idx_i.at[s]], v_mlpi.at[s], sem))
        for cp in copies:
            cp.wait()
        out_sl = pl.ds(base, BPW)
        pltpu.sync_copy(v_mfu, o_mfu.at[out_sl])
        pltpu.sync_copy(v_mfi, o_mfi.at[out_sl])
        pltpu.sync_copy(v_mlpu, o_mlpu.at[out_sl])
        pltpu.sync_copy(v_mlpi, o_mlpi.at[out_sl])

    return k(users, items, mf_u, mf_i, mlp_u, mlp_i)


def _tc_body(gu, gi, mu, mi, w1a, w1b, b1r, w2, b2r, w3, b3r, woa, wob, bor,
             out):
    f32 = jnp.float32
    gmf = gu[:] * gi[:]
    h = jnp.dot(mu[:], w1a[:], preferred_element_type=f32)
    h = h + jnp.dot(mi[:], w1b[:], preferred_element_type=f32)
    h = jnp.maximum(h + b1r[:], 0.0)
    h = jnp.maximum(jnp.dot(h, w2[:], preferred_element_type=f32) + b2r[:], 0.0)
    h = jnp.maximum(jnp.dot(h, w3[:], preferred_element_type=f32) + b3r[:], 0.0)
    logit = (jnp.dot(gmf, woa[:], preferred_element_type=f32)
             + jnp.dot(h, wob[:], preferred_element_type=f32) + bor[:])
    out[:] = jax.nn.sigmoid(logit)


def _tc_mlp(g_u, g_i, m_u, m_i, W1, b1, W2, b2, W3, b3, Wo, bo):
    bs = 2048
    grid = (B // bs,)
    w1a, w1b = W1[:D], W1[D:]
    woa, wob = Wo[:D], Wo[D:]
    b1r = b1.reshape(1, -1)
    b2r = b2.reshape(1, -1)
    b3r = b3.reshape(1, -1)
    bor = bo.reshape(1, 1)

    def full(a):
        return pl.BlockSpec(a.shape, lambda i: (0,) * a.ndim)

    return pl.pallas_call(
        _tc_body,
        grid=grid,
        in_specs=[
            pl.BlockSpec((bs, D), lambda i: (i, 0)),
            pl.BlockSpec((bs, D), lambda i: (i, 0)),
            pl.BlockSpec((bs, D), lambda i: (i, 0)),
            pl.BlockSpec((bs, D), lambda i: (i, 0)),
            full(w1a), full(w1b), full(b1r),
            full(W2), full(b2r),
            full(W3), full(b3r),
            full(woa), full(wob), full(bor),
        ],
        out_specs=pl.BlockSpec((bs, 1), lambda i: (i, 0)),
        out_shape=jax.ShapeDtypeStruct((B, 1), jnp.float32),
    )(g_u, g_i, m_u, m_i, w1a, w1b, b1r, W2, b2r, W3, b3r, woa, wob, bor)


def kernel(users, items, mf_u, mf_i, mlp_u, mlp_i, W1, b1, W2, b2, W3, b3,
           Wo, bo):
    g_u, g_i, m_u, m_i = _sc_gather(users, items, mf_u, mf_i, mlp_u, mlp_i)
    return _tc_mlp(g_u, g_i, m_u, m_i, W1, b1, W2, b2, W3, b3, Wo, bo)
